# baseline (speedup 1.0000x reference)
import functools

import jax
import jax.numpy as jnp
from jax import lax
from jax.experimental import pallas as pl
from jax.experimental.pallas import tpu as pltpu
from jax.experimental.pallas import tpu_sc as plsc

G = 26
V = 100000
D = 16
B = 16384
GD = G * D
H1 = 256
H2 = 128
EPS = 1e-5

_NC = 2
_NS = 16
_NW = _NC * _NS
_NIDX = B * G
_PER_W = _NIDX // _NW
_ROWS = _PER_W // 128
_GK = 8
_NGRP = _ROWS // _GK
_GROWS = _GK * 128


def _gather_body(idx_hbm, tab3_hbm, out_hbm, idx_v, rows_v, sem_g, sem_o):
    wid = lax.axis_index("s") * _NC + lax.axis_index("c")
    rbase = wid * _ROWS
    nbase = wid * _PER_W
    pltpu.sync_copy(idx_hbm.at[pl.ds(rbase, _ROWS)], idx_v)

    def _grp(g, carry):
        s = (g % 2) * _GROWS
        start = nbase + g * _GROWS
        fld = start // B
        b0 = start - fld * B

        @pl.when(g >= 2)
        def _():
            pltpu.make_async_copy(tab3_hbm.at[0, pl.ds(0, _GROWS)],
                                  rows_v.at[pl.ds(0, _GROWS)], sem_o).wait()

        for q in range(_GK):
            r = g * _GK + q
            pltpu.async_copy(tab3_hbm.at[fld].at[idx_v.at[r]],
                             rows_v.at[pl.ds(s + q * 128, 128)], sem_g)
        pltpu.make_async_copy(tab3_hbm.at[0, pl.ds(0, _GROWS)],
                              rows_v.at[pl.ds(0, _GROWS)], sem_g).wait()
        pltpu.async_copy(rows_v.at[pl.ds(s, _GROWS)],
                         out_hbm.at[pl.ds(b0, _GROWS), fld], sem_o)
        return carry

    lax.fori_loop(0, _NGRP, _grp, 0)
    for _ in range(2):
        pltpu.make_async_copy(tab3_hbm.at[0, pl.ds(0, _GROWS)],
                              rows_v.at[pl.ds(0, _GROWS)], sem_o).wait()


@functools.cache
def _mk_gather():
    return functools.partial(
        pl.kernel,
        out_type=jax.ShapeDtypeStruct((B, G, D), jnp.float32),
        mesh=plsc.VectorSubcoreMesh(core_axis_name="c", subcore_axis_name="s",
                                    num_cores=_NC, num_subcores=_NS),
        scratch_types=[
            pltpu.VMEM((_ROWS, 128), jnp.int32),
            pltpu.VMEM((2 * _GROWS, D), jnp.float32),
            pltpu.SemaphoreType.DMA,
            pltpu.SemaphoreType.DMA,
        ],
        compiler_params=pltpu.CompilerParams(use_tc_tiling_on_sc=False),
    )(_gather_body)


TB = 2048
NT = B // TB


def _mlp_body(x_ref, W1_ref, b1_ref, g1a_ref, be1a_ref, g1b_ref, be1b_ref,
              W2_ref, b2_ref, g2a_ref, be2a_ref, g2b_ref, be2b_ref,
              W3_ref, b3_ref, out_ref,
              h1_ref, h2_ref, s1_ref, s2_ref, a1_ref, a2_ref):
    p = pl.program_id(0)
    i = pl.program_id(1)

    def _affine(s_ref, ga, ba, gb, bb, a_ref):
        n = jnp.float32(B)
        m = s_ref[0:1, :] / n
        v = s_ref[1:2, :] / n - m * m
        vy = (ga * ga) * v / (v + EPS)
        scale = ga * gb * lax.rsqrt(v + EPS) * lax.rsqrt(vy + EPS)
        a_ref[0:1, :] = scale
        a_ref[1:2, :] = bb - m * scale

    @pl.when(p == 0)
    def _p0():
        @pl.when(i == 0)
        def _():
            s1_ref[...] = jnp.zeros_like(s1_ref)

        h = jnp.dot(x_ref[...], W1_ref[...],
                    preferred_element_type=jnp.float32) + b1_ref[...]
        h1_ref[pl.ds(i * TB, TB), :] = h
        s1_ref[0:1, :] += jnp.sum(h, axis=0, keepdims=True)
        s1_ref[1:2, :] += jnp.sum(h * h, axis=0, keepdims=True)

        @pl.when(i == NT - 1)
        def _():
            _affine(s1_ref, g1a_ref[...], be1a_ref[...],
                    g1b_ref[...], be1b_ref[...], a1_ref)

    @pl.when(p == 1)
    def _p1():
        @pl.when(i == 0)
        def _():
            s2_ref[...] = jnp.zeros_like(s2_ref)

        h = h1_ref[pl.ds(i * TB, TB), :]
        y = jnp.maximum(h * a1_ref[0:1, :] + a1_ref[1:2, :], 0.0)
        h2 = jnp.dot(y, W2_ref[...],
                     preferred_element_type=jnp.float32) + b2_ref[...]
        h2_ref[pl.ds(i * TB, TB), :] = h2
        s2_ref[0:1, :] += jnp.sum(h2, axis=0, keepdims=True)
        s2_ref[1:2, :] += jnp.sum(h2 * h2, axis=0, keepdims=True)

        @pl.when(i == NT - 1)
        def _():
            _affine(s2_ref, g2a_ref[...], be2a_ref[...],
                    g2b_ref[...], be2b_ref[...], a2_ref)

    @pl.when(p == 2)
    def _p2():
        h = h2_ref[pl.ds(i * TB, TB), :]
        y = jnp.maximum(h * a2_ref[0:1, :] + a2_ref[1:2, :], 0.0)
        z = jnp.dot(y, W3_ref[...],
                    preferred_element_type=jnp.float32) + b3_ref[...]
        out_ref[...] = jax.nn.sigmoid(z)


def _mk_mlp():
    def full(shape):
        return pl.BlockSpec(shape, lambda p, i: tuple(0 for _ in shape))

    return pl.pallas_call(
        _mlp_body,
        grid=(3, NT),
        in_specs=[
            pl.BlockSpec((TB, GD), lambda p, i: (jnp.where(p == 0, i, 0), 0)),
            full((GD, H1)), full((1, H1)), full((1, H1)), full((1, H1)),
            full((1, H1)), full((1, H1)),
            full((H1, H2)), full((1, H2)), full((1, H2)), full((1, H2)),
            full((1, H2)), full((1, H2)),
            full((H2, 1)), full((1, 1)),
        ],
        out_specs=pl.BlockSpec((TB, 1), lambda p, i: (i, 0)),
        out_shape=jax.ShapeDtypeStruct((B, 1), jnp.float32),
        scratch_shapes=[
            pltpu.VMEM((B, H1), jnp.float32),
            pltpu.VMEM((B, H2), jnp.float32),
            pltpu.VMEM((2, H1), jnp.float32),
            pltpu.VMEM((2, H2), jnp.float32),
            pltpu.VMEM((2, H1), jnp.float32),
            pltpu.VMEM((2, H2), jnp.float32),
        ],
    )


_mlp = _mk_mlp()


def kernel(idx, tables, W1, b1, g1a, be1a, g1b, be1b, W2, b2, g2a, be2a,
           g2b, be2b, W3, b3):
    idx2d = idx.T.reshape(_NIDX // 128, 128)
    x = _mk_gather()(idx2d, tables).reshape(B, GD)
    r = lambda a: a.reshape(1, -1)
    return _mlp(x, W1, r(b1), r(g1a), r(be1a), r(g1b), r(be1b),
                W2, r(b2), r(g2a), r(be2a), r(g2b), r(be2b), W3, r(b3))


# speedup vs baseline: 3.2439x; 3.2439x over previous
import functools

import jax
import jax.numpy as jnp
from jax import lax
from jax.experimental import pallas as pl
from jax.experimental.pallas import tpu as pltpu
from jax.experimental.pallas import tpu_sc as plsc

G = 26
V = 100000
D = 16
B = 16384
GD = G * D
H1 = 256
H2 = 128
EPS = 1e-5

_NQ = 98
VP = _NQ * 1024
_GB = 4


def _detile_body(tabT_ref, out_ref):
    out_ref[...] = tabT_ref[...].reshape(128, 1024).T


_detile = pl.pallas_call(
    _detile_body,
    grid=(_GB, _NQ),
    in_specs=[pl.BlockSpec((8, D, 1024), lambda gb, q: (gb, 0, q))],
    out_specs=pl.BlockSpec((1024, 128), lambda gb, q: (gb * _NQ + q, 0)),
    out_shape=jax.ShapeDtypeStruct((_GB * _NQ * 1024, 128), jnp.float32),
)


_NC = 2
_NS = 16
_NW = _NC * _NS
_NIDX = B * G
_PER_W = _NIDX // _NW
_ROWS = _PER_W // 128
_GK = 8
_NGRP = _ROWS // _GK
_GROWS = _GK * 128


def _gather_body(idx_hbm, tab3_hbm, out_hbm, idx_v, rows_v, sem_g, sem_o):
    wid = lax.axis_index("s") * _NC + lax.axis_index("c")
    rbase = wid * _ROWS
    nbase = wid * _PER_W
    pltpu.sync_copy(idx_hbm.at[pl.ds(rbase, _ROWS)], idx_v)

    def _fix(r, carry):
        gg = ((rbase + r) // 128) & 7
        for k in range(8):
            v = idx_v[r, pl.ds(k * 16, 16)]
            idx_v[r, pl.ds(k * 16, 16)] = (v << 3) + gg
        return carry

    lax.fori_loop(0, _ROWS, _fix, 0)

    def _grp(g, carry):
        s = (g % 2) * _GROWS
        start = nbase + g * _GROWS
        fld = start // B
        b0 = start - fld * B

        @pl.when(g >= 2)
        def _():
            pltpu.make_async_copy(tab3_hbm.at[0, pl.ds(0, _GROWS)],
                                  rows_v.at[pl.ds(0, _GROWS)], sem_o).wait()

        for q in range(_GK):
            r = g * _GK + q
            pltpu.async_copy(tab3_hbm.at[fld // 8].at[idx_v.at[r]],
                             rows_v.at[pl.ds(s + q * 128, 128)], sem_g)
        pltpu.make_async_copy(tab3_hbm.at[0, pl.ds(0, _GROWS)],
                              rows_v.at[pl.ds(0, _GROWS)], sem_g).wait()
        pltpu.async_copy(rows_v.at[pl.ds(s, _GROWS)],
                         out_hbm.at[pl.ds(b0, _GROWS), pl.ds(fld * D, D)],
                         sem_o)
        return carry

    lax.fori_loop(0, _NGRP, _grp, 0)
    for _ in range(2):
        pltpu.make_async_copy(tab3_hbm.at[0, pl.ds(0, _GROWS)],
                              rows_v.at[pl.ds(0, _GROWS)], sem_o).wait()


@functools.cache
def _mk_gather():
    return functools.partial(
        pl.kernel,
        out_type=jax.ShapeDtypeStruct((B, GD), jnp.float32),
        mesh=plsc.VectorSubcoreMesh(core_axis_name="c", subcore_axis_name="s",
                                    num_cores=_NC, num_subcores=_NS),
        scratch_types=[
            pltpu.VMEM((_ROWS, 128), jnp.int32),
            pltpu.VMEM((2 * _GROWS, D), jnp.float32),
            pltpu.SemaphoreType.DMA,
            pltpu.SemaphoreType.DMA,
        ],
        compiler_params=pltpu.CompilerParams(use_tc_tiling_on_sc=False),
    )(_gather_body)


TB = 2048
NT = B // TB


def _mlp_body(x_ref, W1_ref, b1_ref, g1a_ref, be1a_ref, g1b_ref, be1b_ref,
              W2_ref, b2_ref, g2a_ref, be2a_ref, g2b_ref, be2b_ref,
              W3_ref, b3_ref, out_ref,
              h1_ref, h2_ref, s1_ref, s2_ref, a1_ref, a2_ref):
    p = pl.program_id(0)
    i = pl.program_id(1)

    def _affine(s_ref, ga, ba, gb, bb, a_ref):
        n = jnp.float32(B)
        m = s_ref[0:1, :] / n
        v = s_ref[1:2, :] / n - m * m
        vy = (ga * ga) * v / (v + EPS)
        scale = ga * gb * lax.rsqrt(v + EPS) * lax.rsqrt(vy + EPS)
        a_ref[0:1, :] = scale
        a_ref[1:2, :] = bb - m * scale

    @pl.when(p == 0)
    def _p0():
        @pl.when(i == 0)
        def _():
            s1_ref[...] = jnp.zeros_like(s1_ref)

        h = jnp.dot(x_ref[...], W1_ref[...],
                    preferred_element_type=jnp.float32) + b1_ref[...]
        h1_ref[pl.ds(i * TB, TB), :] = h
        s1_ref[0:1, :] += jnp.sum(h, axis=0, keepdims=True)
        s1_ref[1:2, :] += jnp.sum(h * h, axis=0, keepdims=True)

        @pl.when(i == NT - 1)
        def _():
            _affine(s1_ref, g1a_ref[...], be1a_ref[...],
                    g1b_ref[...], be1b_ref[...], a1_ref)

    @pl.when(p == 1)
    def _p1():
        @pl.when(i == 0)
        def _():
            s2_ref[...] = jnp.zeros_like(s2_ref)

        h = h1_ref[pl.ds(i * TB, TB), :]
        y = jnp.maximum(h * a1_ref[0:1, :] + a1_ref[1:2, :], 0.0)
        h2 = jnp.dot(y, W2_ref[...],
                     preferred_element_type=jnp.float32) + b2_ref[...]
        h2_ref[pl.ds(i * TB, TB), :] = h2
        s2_ref[0:1, :] += jnp.sum(h2, axis=0, keepdims=True)
        s2_ref[1:2, :] += jnp.sum(h2 * h2, axis=0, keepdims=True)

        @pl.when(i == NT - 1)
        def _():
            _affine(s2_ref, g2a_ref[...], be2a_ref[...],
                    g2b_ref[...], be2b_ref[...], a2_ref)

    @pl.when(p == 2)
    def _p2():
        h = h2_ref[pl.ds(i * TB, TB), :]
        y = jnp.maximum(h * a2_ref[0:1, :] + a2_ref[1:2, :], 0.0)
        z = jnp.dot(y, W3_ref[...],
                    preferred_element_type=jnp.float32) + b3_ref[...]
        out_ref[...] = jax.nn.sigmoid(z)


def _mk_mlp():
    def full(shape):
        return pl.BlockSpec(shape, lambda p, i: tuple(0 for _ in shape))

    return pl.pallas_call(
        _mlp_body,
        grid=(3, NT),
        in_specs=[
            pl.BlockSpec((TB, GD), lambda p, i: (jnp.where(p == 0, i, 0), 0)),
            full((GD, H1)), full((1, H1)), full((1, H1)), full((1, H1)),
            full((1, H1)), full((1, H1)),
            full((H1, H2)), full((1, H2)), full((1, H2)), full((1, H2)),
            full((1, H2)), full((1, H2)),
            full((H2, 1)), full((1, 1)),
        ],
        out_specs=pl.BlockSpec((TB, 1), lambda p, i: (i, 0)),
        out_shape=jax.ShapeDtypeStruct((B, 1), jnp.float32),
        scratch_shapes=[
            pltpu.VMEM((B, H1), jnp.float32),
            pltpu.VMEM((B, H2), jnp.float32),
            pltpu.VMEM((2, H1), jnp.float32),
            pltpu.VMEM((2, H2), jnp.float32),
            pltpu.VMEM((2, H1), jnp.float32),
            pltpu.VMEM((2, H2), jnp.float32),
        ],
        compiler_params=pltpu.CompilerParams(
            vmem_limit_bytes=58 * 1024 * 1024),
    )


_mlp = _mk_mlp()


def kernel(idx, tables, W1, b1, g1a, be1a, g1b, be1b, W2, b2, g2a, be2a,
           g2b, be2b, W3, b3):
    idx2d = idx.T.reshape(_NIDX // 128, 128)
    tab_lin = _detile(jnp.transpose(tables, (0, 2, 1))).reshape(_GB, 8 * VP, D)
    x = _mk_gather()(idx2d, tab_lin)
    r = lambda a: a.reshape(1, -1)
    return _mlp(x, W1, r(b1), r(g1a), r(be1a), r(g1b), r(be1b),
                W2, r(b2), r(g2a), r(be2a), r(g2b), r(be2b), W3, r(b3))


# speedup vs baseline: 5.3729x; 1.6563x over previous
import functools

import jax
import jax.numpy as jnp
from jax import lax
from jax.experimental import pallas as pl
from jax.experimental.pallas import tpu as pltpu
from jax.experimental.pallas import tpu_sc as plsc

G = 26
V = 100000
D = 16
B = 16384
GD = G * D
H1 = 256
H2 = 128
EPS = 1e-5

_QW = 8192
_NQ = 13
VP = _NQ * _QW
_GB = 4


def _detile_body(tabT_ref, out_ref):
    out_ref[...] = tabT_ref[...].reshape(128, _QW).T


_detile = pl.pallas_call(
    _detile_body,
    grid=(_GB, _NQ),
    in_specs=[pl.BlockSpec((8, D, _QW), lambda gb, q: (gb, 0, q))],
    out_specs=pl.BlockSpec((_QW, 128), lambda gb, q: (gb * _NQ + q, 0)),
    out_shape=jax.ShapeDtypeStruct((_GB * _NQ * _QW, 128), jnp.float32),
    compiler_params=pltpu.CompilerParams(
        vmem_limit_bytes=58 * 1024 * 1024),
)


_NC = 2
_NS = 16
_NW = _NC * _NS
_NIDX = B * G
_PER_W = _NIDX // _NW
_ROWS = _PER_W // 128
_GK = 8
_NGRP = _ROWS // _GK
_GROWS = _GK * 128


def _gather_body(idx_hbm, tab3_hbm, out_hbm, idx_v, rows_v, sem_g, sem_o):
    wid = lax.axis_index("s") * _NC + lax.axis_index("c")
    rbase = wid * _ROWS
    nbase = wid * _PER_W
    pltpu.sync_copy(idx_hbm.at[pl.ds(rbase, _ROWS)], idx_v)

    def _fix(r, carry):
        gg = ((rbase + r) // 128) & 7
        for k in range(8):
            v = idx_v[r, pl.ds(k * 16, 16)]
            idx_v[r, pl.ds(k * 16, 16)] = (v << 3) + gg
        return carry

    lax.fori_loop(0, _ROWS, _fix, 0)

    def _grp(g, carry):
        s = (g % 2) * _GROWS
        start = nbase + g * _GROWS
        fld = start // B
        b0 = start - fld * B

        @pl.when(g >= 2)
        def _():
            pltpu.make_async_copy(tab3_hbm.at[0, pl.ds(0, _GROWS)],
                                  rows_v.at[pl.ds(0, _GROWS)], sem_o).wait()

        for q in range(_GK):
            r = g * _GK + q
            pltpu.async_copy(tab3_hbm.at[fld // 8].at[idx_v.at[r]],
                             rows_v.at[pl.ds(s + q * 128, 128)], sem_g)
        pltpu.make_async_copy(tab3_hbm.at[0, pl.ds(0, _GROWS)],
                              rows_v.at[pl.ds(0, _GROWS)], sem_g).wait()
        pltpu.async_copy(rows_v.at[pl.ds(s, _GROWS)],
                         out_hbm.at[pl.ds(b0, _GROWS), pl.ds(fld * D, D)],
                         sem_o)
        return carry

    lax.fori_loop(0, _NGRP, _grp, 0)
    for _ in range(2):
        pltpu.make_async_copy(tab3_hbm.at[0, pl.ds(0, _GROWS)],
                              rows_v.at[pl.ds(0, _GROWS)], sem_o).wait()


@functools.cache
def _mk_gather():
    return functools.partial(
        pl.kernel,
        out_type=jax.ShapeDtypeStruct((B, GD), jnp.float32),
        mesh=plsc.VectorSubcoreMesh(core_axis_name="c", subcore_axis_name="s",
                                    num_cores=_NC, num_subcores=_NS),
        scratch_types=[
            pltpu.VMEM((_ROWS, 128), jnp.int32),
            pltpu.VMEM((2 * _GROWS, D), jnp.float32),
            pltpu.SemaphoreType.DMA,
            pltpu.SemaphoreType.DMA,
        ],
        compiler_params=pltpu.CompilerParams(use_tc_tiling_on_sc=False),
    )(_gather_body)


TB = 2048
NT = B // TB


def _mlp_body(x_ref, W1_ref, b1_ref, g1a_ref, be1a_ref, g1b_ref, be1b_ref,
              W2_ref, b2_ref, g2a_ref, be2a_ref, g2b_ref, be2b_ref,
              W3_ref, b3_ref, out_ref,
              h1_ref, h2_ref, s1_ref, s2_ref, a1_ref, a2_ref):
    p = pl.program_id(0)
    i = pl.program_id(1)

    def _affine(s_ref, ga, ba, gb, bb, a_ref):
        n = jnp.float32(B)
        m = s_ref[0:1, :] / n
        v = s_ref[1:2, :] / n - m * m
        vy = (ga * ga) * v / (v + EPS)
        scale = ga * gb * lax.rsqrt(v + EPS) * lax.rsqrt(vy + EPS)
        a_ref[0:1, :] = scale
        a_ref[1:2, :] = bb - m * scale

    @pl.when(p == 0)
    def _p0():
        @pl.when(i == 0)
        def _():
            s1_ref[...] = jnp.zeros_like(s1_ref)

        h = jnp.dot(x_ref[...], W1_ref[...],
                    preferred_element_type=jnp.float32) + b1_ref[...]
        h1_ref[pl.ds(i * TB, TB), :] = h
        s1_ref[0:1, :] += jnp.sum(h, axis=0, keepdims=True)
        s1_ref[1:2, :] += jnp.sum(h * h, axis=0, keepdims=True)

        @pl.when(i == NT - 1)
        def _():
            _affine(s1_ref, g1a_ref[...], be1a_ref[...],
                    g1b_ref[...], be1b_ref[...], a1_ref)

    @pl.when(p == 1)
    def _p1():
        @pl.when(i == 0)
        def _():
            s2_ref[...] = jnp.zeros_like(s2_ref)

        h = h1_ref[pl.ds(i * TB, TB), :]
        y = jnp.maximum(h * a1_ref[0:1, :] + a1_ref[1:2, :], 0.0)
        h2 = jnp.dot(y, W2_ref[...],
                     preferred_element_type=jnp.float32) + b2_ref[...]
        h2_ref[pl.ds(i * TB, TB), :] = h2
        s2_ref[0:1, :] += jnp.sum(h2, axis=0, keepdims=True)
        s2_ref[1:2, :] += jnp.sum(h2 * h2, axis=0, keepdims=True)

        @pl.when(i == NT - 1)
        def _():
            _affine(s2_ref, g2a_ref[...], be2a_ref[...],
                    g2b_ref[...], be2b_ref[...], a2_ref)

    @pl.when(p == 2)
    def _p2():
        h = h2_ref[pl.ds(i * TB, TB), :]
        y = jnp.maximum(h * a2_ref[0:1, :] + a2_ref[1:2, :], 0.0)
        z = jnp.dot(y, W3_ref[...],
                    preferred_element_type=jnp.float32) + b3_ref[...]
        out_ref[...] = jax.nn.sigmoid(z)


def _mk_mlp():
    def full(shape):
        return pl.BlockSpec(shape, lambda p, i: tuple(0 for _ in shape))

    return pl.pallas_call(
        _mlp_body,
        grid=(3, NT),
        in_specs=[
            pl.BlockSpec((TB, GD), lambda p, i: (jnp.where(p == 0, i, 0), 0)),
            full((GD, H1)), full((1, H1)), full((1, H1)), full((1, H1)),
            full((1, H1)), full((1, H1)),
            full((H1, H2)), full((1, H2)), full((1, H2)), full((1, H2)),
            full((1, H2)), full((1, H2)),
            full((H2, 1)), full((1, 1)),
        ],
        out_specs=pl.BlockSpec((TB, 1), lambda p, i: (i, 0)),
        out_shape=jax.ShapeDtypeStruct((B, 1), jnp.float32),
        scratch_shapes=[
            pltpu.VMEM((B, H1), jnp.float32),
            pltpu.VMEM((B, H2), jnp.float32),
            pltpu.VMEM((2, H1), jnp.float32),
            pltpu.VMEM((2, H2), jnp.float32),
            pltpu.VMEM((2, H1), jnp.float32),
            pltpu.VMEM((2, H2), jnp.float32),
        ],
        compiler_params=pltpu.CompilerParams(
            vmem_limit_bytes=58 * 1024 * 1024),
    )


_mlp = _mk_mlp()


def kernel(idx, tables, W1, b1, g1a, be1a, g1b, be1b, W2, b2, g2a, be2a,
           g2b, be2b, W3, b3):
    idx2d = idx.T.reshape(_NIDX // 128, 128)
    tab_lin = _detile(jnp.transpose(tables, (0, 2, 1))).reshape(_GB, 8 * VP, D)
    x = _mk_gather()(idx2d, tab_lin)
    r = lambda a: a.reshape(1, -1)
    return _mlp(x, W1, r(b1), r(g1a), r(be1a), r(g1b), r(be1b),
                W2, r(b2), r(g2a), r(be2a), r(g2b), r(be2b), W3, r(b3))


# speedup vs baseline: 6.1607x; 1.1466x over previous
import functools

import jax
import jax.numpy as jnp
from jax import lax
from jax.experimental import pallas as pl
from jax.experimental.pallas import tpu as pltpu
from jax.experimental.pallas import tpu_sc as plsc

G = 26
V = 100000
D = 16
B = 16384
GD = G * D
H1 = 256
H2 = 128
EPS = 1e-5

_QW = 16384
_NQ = 7
VP = _NQ * _QW
_GB = 4


def _detile_body(tabT_ref, out_ref):
    out_ref[...] = tabT_ref[...].reshape(128, _QW).T


_detile = pl.pallas_call(
    _detile_body,
    grid=(_GB, _NQ),
    in_specs=[pl.BlockSpec((8, D, _QW), lambda gb, q: (gb, 0, q))],
    out_specs=pl.BlockSpec((_QW, 128), lambda gb, q: (gb * _NQ + q, 0)),
    out_shape=jax.ShapeDtypeStruct((_GB * _NQ * _QW, 128), jnp.float32),
    compiler_params=pltpu.CompilerParams(
        vmem_limit_bytes=58 * 1024 * 1024),
)


_NC = 2
_NS = 16
_NW = _NC * _NS
_NIDX = B * G
_PER_W = _NIDX // _NW
_ROWS = _PER_W // 128
_GK = 8
_NGRP = _ROWS // _GK
_GROWS = _GK * 128


def _gather_body(idx_hbm, tab3_hbm, o0, o1, o2, o3, idx_v, rows_v,
                 sem_g, sem_o):
    wid = lax.axis_index("s") * _NC + lax.axis_index("c")
    rbase = wid * _ROWS
    nbase = wid * _PER_W
    pltpu.sync_copy(idx_hbm.at[pl.ds(rbase, _ROWS)], idx_v)

    def _fix(r, carry):
        gg = ((rbase + r) // 128) & 7
        for k in range(8):
            v = idx_v[r, pl.ds(k * 16, 16)]
            idx_v[r, pl.ds(k * 16, 16)] = (v << 3) + gg
        return carry

    lax.fori_loop(0, _ROWS, _fix, 0)

    def _grp(g, carry):
        s = (g % 2) * _GROWS
        start = nbase + g * _GROWS
        fld = start // B
        b0 = start - fld * B

        @pl.when(g >= 2)
        def _():
            pltpu.make_async_copy(tab3_hbm.at[0, pl.ds(0, _GROWS)],
                                  rows_v.at[pl.ds(0, _GROWS)], sem_o).wait()

        for q in range(_GK):
            r = g * _GK + q
            pltpu.async_copy(tab3_hbm.at[fld // 8].at[idx_v.at[r]],
                             rows_v.at[pl.ds(s + q * 128, 128)], sem_g)
        pltpu.make_async_copy(tab3_hbm.at[0, pl.ds(0, _GROWS)],
                              rows_v.at[pl.ds(0, _GROWS)], sem_g).wait()
        lane = (fld % 8) * D
        for ob, o_hbm in enumerate((o0, o1, o2, o3)):
            @pl.when(fld // 8 == ob)
            def _(o_hbm=o_hbm):
                pltpu.async_copy(rows_v.at[pl.ds(s, _GROWS)],
                                 o_hbm.at[pl.ds(b0, _GROWS),
                                          pl.ds(lane, D)],
                                 sem_o)
        return carry

    lax.fori_loop(0, _NGRP, _grp, 0)
    for _ in range(2):
        pltpu.make_async_copy(tab3_hbm.at[0, pl.ds(0, _GROWS)],
                              rows_v.at[pl.ds(0, _GROWS)], sem_o).wait()


@functools.cache
def _mk_gather():
    return functools.partial(
        pl.kernel,
        out_type=[jax.ShapeDtypeStruct((B, 128), jnp.float32)
                  for _ in range(4)],
        mesh=plsc.VectorSubcoreMesh(core_axis_name="c", subcore_axis_name="s",
                                    num_cores=_NC, num_subcores=_NS),
        scratch_types=[
            pltpu.VMEM((_ROWS, 128), jnp.int32),
            pltpu.VMEM((2 * _GROWS, D), jnp.float32),
            pltpu.SemaphoreType.DMA,
            pltpu.SemaphoreType.DMA,
        ],
        compiler_params=pltpu.CompilerParams(use_tc_tiling_on_sc=False),
    )(_gather_body)


TB = 2048
NT = B // TB


def _mlp_body(x0_ref, x1_ref, x2_ref, x3_ref,
              W1_ref, b1_ref, g1a_ref, be1a_ref, g1b_ref, be1b_ref,
              W2_ref, b2_ref, g2a_ref, be2a_ref, g2b_ref, be2b_ref,
              W3_ref, b3_ref, out_ref,
              h1_ref, h2_ref, s1_ref, s2_ref, a1_ref, a2_ref):
    p = pl.program_id(0)
    i = pl.program_id(1)

    def _affine(s_ref, ga, ba, gb, bb, a_ref):
        n = jnp.float32(B)
        m = s_ref[0:1, :] / n
        v = s_ref[1:2, :] / n - m * m
        vy = (ga * ga) * v / (v + EPS)
        scale = ga * gb * lax.rsqrt(v + EPS) * lax.rsqrt(vy + EPS)
        a_ref[0:1, :] = scale
        a_ref[1:2, :] = bb - m * scale

    @pl.when(p == 0)
    def _p0():
        @pl.when(i == 0)
        def _():
            s1_ref[...] = jnp.zeros_like(s1_ref)

        h = (jnp.dot(x0_ref[...], W1_ref[0:128, :],
                     preferred_element_type=jnp.float32)
             + jnp.dot(x1_ref[...], W1_ref[128:256, :],
                       preferred_element_type=jnp.float32)
             + jnp.dot(x2_ref[...], W1_ref[256:384, :],
                       preferred_element_type=jnp.float32)
             + jnp.dot(x3_ref[:, 0:32], W1_ref[384:416, :],
                       preferred_element_type=jnp.float32)
             + b1_ref[...])
        h1_ref[pl.ds(i * TB, TB), :] = h
        s1_ref[0:1, :] += jnp.sum(h, axis=0, keepdims=True)
        s1_ref[1:2, :] += jnp.sum(h * h, axis=0, keepdims=True)

        @pl.when(i == NT - 1)
        def _():
            _affine(s1_ref, g1a_ref[...], be1a_ref[...],
                    g1b_ref[...], be1b_ref[...], a1_ref)

    @pl.when(p == 1)
    def _p1():
        @pl.when(i == 0)
        def _():
            s2_ref[...] = jnp.zeros_like(s2_ref)

        h = h1_ref[pl.ds(i * TB, TB), :]
        y = jnp.maximum(h * a1_ref[0:1, :] + a1_ref[1:2, :], 0.0)
        h2 = jnp.dot(y, W2_ref[...],
                     preferred_element_type=jnp.float32) + b2_ref[...]
        h2_ref[pl.ds(i * TB, TB), :] = h2
        s2_ref[0:1, :] += jnp.sum(h2, axis=0, keepdims=True)
        s2_ref[1:2, :] += jnp.sum(h2 * h2, axis=0, keepdims=True)

        @pl.when(i == NT - 1)
        def _():
            _affine(s2_ref, g2a_ref[...], be2a_ref[...],
                    g2b_ref[...], be2b_ref[...], a2_ref)

    @pl.when(p == 2)
    def _p2():
        h = h2_ref[pl.ds(i * TB, TB), :]
        y = jnp.maximum(h * a2_ref[0:1, :] + a2_ref[1:2, :], 0.0)
        z = jnp.dot(y, W3_ref[...],
                    preferred_element_type=jnp.float32) + b3_ref[...]
        out_ref[...] = jax.nn.sigmoid(z)


def _mk_mlp():
    def full(shape):
        return pl.BlockSpec(shape, lambda p, i: tuple(0 for _ in shape))

    return pl.pallas_call(
        _mlp_body,
        grid=(3, NT),
        in_specs=[
            pl.BlockSpec((TB, 128), lambda p, i: (jnp.where(p == 0, i, 0), 0)),
            pl.BlockSpec((TB, 128), lambda p, i: (jnp.where(p == 0, i, 0), 0)),
            pl.BlockSpec((TB, 128), lambda p, i: (jnp.where(p == 0, i, 0), 0)),
            pl.BlockSpec((TB, 128), lambda p, i: (jnp.where(p == 0, i, 0), 0)),
            full((GD, H1)), full((1, H1)), full((1, H1)), full((1, H1)),
            full((1, H1)), full((1, H1)),
            full((H1, H2)), full((1, H2)), full((1, H2)), full((1, H2)),
            full((1, H2)), full((1, H2)),
            full((H2, 1)), full((1, 1)),
        ],
        out_specs=pl.BlockSpec((TB, 1), lambda p, i: (i, 0)),
        out_shape=jax.ShapeDtypeStruct((B, 1), jnp.float32),
        scratch_shapes=[
            pltpu.VMEM((B, H1), jnp.float32),
            pltpu.VMEM((B, H2), jnp.float32),
            pltpu.VMEM((2, H1), jnp.float32),
            pltpu.VMEM((2, H2), jnp.float32),
            pltpu.VMEM((2, H1), jnp.float32),
            pltpu.VMEM((2, H2), jnp.float32),
        ],
        compiler_params=pltpu.CompilerParams(
            vmem_limit_bytes=58 * 1024 * 1024),
    )


_mlp = _mk_mlp()


def kernel(idx, tables, W1, b1, g1a, be1a, g1b, be1b, W2, b2, g2a, be2a,
           g2b, be2b, W3, b3):
    idx2d = idx.T.reshape(_NIDX // 128, 128)
    tab_lin = _detile(jnp.transpose(tables, (0, 2, 1))).reshape(_GB, 8 * VP, D)
    x0, x1, x2, x3 = _mk_gather()(idx2d, tab_lin)
    r = lambda a: a.reshape(1, -1)
    return _mlp(x0, x1, x2, x3, W1, r(b1), r(g1a), r(be1a), r(g1b), r(be1b),
                W2, r(b2), r(g2a), r(be2a), r(g2b), r(be2b), W3, r(b3))


# speedup vs baseline: 6.4755x; 1.0511x over previous
import functools

import jax
import jax.numpy as jnp
from jax import lax
from jax.experimental import pallas as pl
from jax.experimental.pallas import tpu as pltpu
from jax.experimental.pallas import tpu_sc as plsc

G = 26
V = 100000
D = 16
B = 16384
GD = G * D
H1 = 256
H2 = 128
EPS = 1e-5

_QW = 12544
_NQ = 8
VP = _NQ * _QW
_GB = 4


def _detile_body(tabT_ref, out_ref):
    out_ref[...] = tabT_ref[...].reshape(128, _QW).T


_detile = pl.pallas_call(
    _detile_body,
    grid=(_GB, _NQ),
    in_specs=[pl.BlockSpec((8, D, _QW), lambda gb, q: (gb, 0, q))],
    out_specs=pl.BlockSpec((_QW, 128), lambda gb, q: (gb * _NQ + q, 0)),
    out_shape=jax.ShapeDtypeStruct((_GB * _NQ * _QW, 128), jnp.float32),
    compiler_params=pltpu.CompilerParams(
        vmem_limit_bytes=58 * 1024 * 1024),
)


_NC = 2
_NS = 16
_NW = _NC * _NS
_NIDX = B * G
_PER_W = _NIDX // _NW
_ROWS = _PER_W // 128
_GK = 8
_NGRP = _ROWS // _GK
_GROWS = _GK * 128


def _gather_body(idx_hbm, tab3_hbm, o0, o1, o2, o3, idx_v, rows_v,
                 sem_g, sem_o):
    wid = lax.axis_index("s") * _NC + lax.axis_index("c")
    rbase = wid * _ROWS
    nbase = wid * _PER_W
    pltpu.sync_copy(idx_hbm.at[pl.ds(rbase, _ROWS)], idx_v)

    def _fix(r, carry):
        gg = ((rbase + r) // 128) & 7
        for k in range(8):
            v = idx_v[r, pl.ds(k * 16, 16)]
            idx_v[r, pl.ds(k * 16, 16)] = (v << 3) + gg
        return carry

    lax.fori_loop(0, _ROWS, _fix, 0)

    def _grp(g, carry):
        s = (g % 2) * _GROWS
        start = nbase + g * _GROWS
        fld = start // B
        b0 = start - fld * B

        @pl.when(g >= 2)
        def _():
            pltpu.make_async_copy(tab3_hbm.at[0, pl.ds(0, _GROWS)],
                                  rows_v.at[pl.ds(0, _GROWS)], sem_o).wait()

        for q in range(_GK):
            r = g * _GK + q
            pltpu.async_copy(tab3_hbm.at[fld // 8].at[idx_v.at[r]],
                             rows_v.at[pl.ds(s + q * 128, 128)], sem_g)
        pltpu.make_async_copy(tab3_hbm.at[0, pl.ds(0, _GROWS)],
                              rows_v.at[pl.ds(0, _GROWS)], sem_g).wait()
        lane = (fld % 8) * D
        for ob, o_hbm in enumerate((o0, o1, o2, o3)):
            @pl.when(fld // 8 == ob)
            def _(o_hbm=o_hbm):
                pltpu.async_copy(rows_v.at[pl.ds(s, _GROWS)],
                                 o_hbm.at[pl.ds(b0, _GROWS),
                                          pl.ds(lane, D)],
                                 sem_o)
        return carry

    lax.fori_loop(0, _NGRP, _grp, 0)
    for _ in range(2):
        pltpu.make_async_copy(tab3_hbm.at[0, pl.ds(0, _GROWS)],
                              rows_v.at[pl.ds(0, _GROWS)], sem_o).wait()


@functools.cache
def _mk_gather():
    return functools.partial(
        pl.kernel,
        out_type=[jax.ShapeDtypeStruct((B, 128), jnp.float32)
                  for _ in range(4)],
        mesh=plsc.VectorSubcoreMesh(core_axis_name="c", subcore_axis_name="s",
                                    num_cores=_NC, num_subcores=_NS),
        scratch_types=[
            pltpu.VMEM((_ROWS, 128), jnp.int32),
            pltpu.VMEM((2 * _GROWS, D), jnp.float32),
            pltpu.SemaphoreType.DMA,
            pltpu.SemaphoreType.DMA,
        ],
        compiler_params=pltpu.CompilerParams(use_tc_tiling_on_sc=False),
    )(_gather_body)


TB = 2048
NT = B // TB


def _mlp_body(x0_ref, x1_ref, x2_ref, x3_ref,
              W1_ref, b1_ref, g1a_ref, be1a_ref, g1b_ref, be1b_ref,
              W2_ref, b2_ref, g2a_ref, be2a_ref, g2b_ref, be2b_ref,
              W3_ref, b3_ref, out_ref,
              h1_ref, h2_ref, s1_ref, s2_ref, a1_ref, a2_ref):
    p = pl.program_id(0)
    i = pl.program_id(1)

    def _affine(s_ref, ga, ba, gb, bb, a_ref):
        n = jnp.float32(B)
        m = s_ref[0:1, :] / n
        v = s_ref[1:2, :] / n - m * m
        vy = (ga * ga) * v / (v + EPS)
        scale = ga * gb * lax.rsqrt(v + EPS) * lax.rsqrt(vy + EPS)
        a_ref[0:1, :] = scale
        a_ref[1:2, :] = bb - m * scale

    @pl.when(p == 0)
    def _p0():
        @pl.when(i == 0)
        def _():
            s1_ref[...] = jnp.zeros_like(s1_ref)

        h = (jnp.dot(x0_ref[...], W1_ref[0:128, :],
                     preferred_element_type=jnp.float32)
             + jnp.dot(x1_ref[...], W1_ref[128:256, :],
                       preferred_element_type=jnp.float32)
             + jnp.dot(x2_ref[...], W1_ref[256:384, :],
                       preferred_element_type=jnp.float32)
             + jnp.dot(x3_ref[:, 0:32], W1_ref[384:416, :],
                       preferred_element_type=jnp.float32)
             + b1_ref[...])
        h1_ref[pl.ds(i * TB, TB), :] = h
        s1_ref[0:1, :] += jnp.sum(h, axis=0, keepdims=True)
        s1_ref[1:2, :] += jnp.sum(h * h, axis=0, keepdims=True)

        @pl.when(i == NT - 1)
        def _():
            _affine(s1_ref, g1a_ref[...], be1a_ref[...],
                    g1b_ref[...], be1b_ref[...], a1_ref)

    @pl.when(p == 1)
    def _p1():
        @pl.when(i == 0)
        def _():
            s2_ref[...] = jnp.zeros_like(s2_ref)

        h = h1_ref[pl.ds(i * TB, TB), :]
        y = jnp.maximum(h * a1_ref[0:1, :] + a1_ref[1:2, :], 0.0)
        h2 = jnp.dot(y, W2_ref[...],
                     preferred_element_type=jnp.float32) + b2_ref[...]
        h2_ref[pl.ds(i * TB, TB), :] = h2
        s2_ref[0:1, :] += jnp.sum(h2, axis=0, keepdims=True)
        s2_ref[1:2, :] += jnp.sum(h2 * h2, axis=0, keepdims=True)

        @pl.when(i == NT - 1)
        def _():
            _affine(s2_ref, g2a_ref[...], be2a_ref[...],
                    g2b_ref[...], be2b_ref[...], a2_ref)

    @pl.when(p == 2)
    def _p2():
        h = h2_ref[pl.ds(i * TB, TB), :]
        y = jnp.maximum(h * a2_ref[0:1, :] + a2_ref[1:2, :], 0.0)
        z = jnp.dot(y, W3_ref[...],
                    preferred_element_type=jnp.float32) + b3_ref[...]
        out_ref[...] = jax.nn.sigmoid(z)


def _mk_mlp():
    def full(shape):
        return pl.BlockSpec(shape, lambda p, i: tuple(0 for _ in shape))

    return pl.pallas_call(
        _mlp_body,
        grid=(3, NT),
        in_specs=[
            pl.BlockSpec((TB, 128), lambda p, i: (jnp.where(p == 0, i, 0), 0)),
            pl.BlockSpec((TB, 128), lambda p, i: (jnp.where(p == 0, i, 0), 0)),
            pl.BlockSpec((TB, 128), lambda p, i: (jnp.where(p == 0, i, 0), 0)),
            pl.BlockSpec((TB, 128), lambda p, i: (jnp.where(p == 0, i, 0), 0)),
            full((GD, H1)), full((1, H1)), full((1, H1)), full((1, H1)),
            full((1, H1)), full((1, H1)),
            full((H1, H2)), full((1, H2)), full((1, H2)), full((1, H2)),
            full((1, H2)), full((1, H2)),
            full((H2, 1)), full((1, 1)),
        ],
        out_specs=pl.BlockSpec((TB, 1), lambda p, i: (i, 0)),
        out_shape=jax.ShapeDtypeStruct((B, 1), jnp.float32),
        scratch_shapes=[
            pltpu.VMEM((B, H1), jnp.float32),
            pltpu.VMEM((B, H2), jnp.float32),
            pltpu.VMEM((2, H1), jnp.float32),
            pltpu.VMEM((2, H2), jnp.float32),
            pltpu.VMEM((2, H1), jnp.float32),
            pltpu.VMEM((2, H2), jnp.float32),
        ],
        compiler_params=pltpu.CompilerParams(
            vmem_limit_bytes=58 * 1024 * 1024),
    )


_mlp = _mk_mlp()


def kernel(idx, tables, W1, b1, g1a, be1a, g1b, be1b, W2, b2, g2a, be2a,
           g2b, be2b, W3, b3):
    idx2d = idx.T.reshape(_NIDX // 128, 128)
    tab_lin = _detile(jnp.transpose(tables, (0, 2, 1))).reshape(_GB, 8 * VP, D)
    x0, x1, x2, x3 = _mk_gather()(idx2d, tab_lin)
    r = lambda a: a.reshape(1, -1)
    return _mlp(x0, x1, x2, x3, W1, r(b1), r(g1a), r(be1a), r(g1b), r(be1b),
                W2, r(b2), r(g2a), r(be2a), r(g2b), r(be2b), W3, r(b3))
